# baseline (speedup 1.0000x reference)
import functools

import jax
import jax.numpy as jnp
from jax.experimental import pallas as pl
from jax.experimental.pallas import tpu as pltpu

B, N, T, P, D, O = 16, 2048, 1, 6, 128, 128
ROWS = 512


def _fused_body(embs_ref, vis_ref, wt_ref, b_ref, mask_ref, out_ref):
    combined = embs_ref[:, 0, :]
    scale = jnp.ones((ROWS, 1), jnp.float32)
    for i in range(1, P):
        v = vis_ref[:, i : i + 1]
        combined = combined + embs_ref[:, i, :] * v
        scale = scale + v
    lin = jnp.dot(combined, wt_ref[:, :], preferred_element_type=jnp.float32)
    out_ref[:, :] = (lin + scale * b_ref[0, :]) * mask_ref[:, :]


@jax.jit
def kernel(embs, vis, W, b, masks):
    rows = B * N
    embs3 = embs.reshape(rows, P, D)
    vis2 = vis.reshape(rows, P)
    maskf = masks.reshape(rows, 1).astype(jnp.float32)
    wt = W.T
    b2 = b.reshape(1, O)
    grid = (rows // ROWS,)
    out = pl.pallas_call(
        _fused_body,
        grid=grid,
        in_specs=[
            pl.BlockSpec((ROWS, P, D), lambda i: (i, 0, 0)),
            pl.BlockSpec((ROWS, P), lambda i: (i, 0)),
            pl.BlockSpec((D, O), lambda i: (0, 0)),
            pl.BlockSpec((1, O), lambda i: (0, 0)),
            pl.BlockSpec((ROWS, 1), lambda i: (i, 0)),
        ],
        out_specs=pl.BlockSpec((ROWS, O), lambda i: (i, 0)),
        out_shape=jax.ShapeDtypeStruct((rows, O), jnp.float32),
        compiler_params=pltpu.CompilerParams(
            dimension_semantics=("arbitrary",),
        ),
    )(embs3, vis2, wt, b2, maskf)
    return out.reshape(B, N, O)
